# Initial kernel scaffold; baseline (speedup 1.0000x reference)
import jax
import jax.numpy as jnp
from jax.experimental import pallas as pl


def kernel(x, edge_index, true_y, mask, W_proj_gate, b_proj_gate, a_src_gate, a_trg_gate, W_skip, b_skip, W_cheb0, W_cheb1, b_cheb, W_gat, a_src_gat, a_trg_gat, b_gat, gumbel_noise):
    raise NotImplementedError("write your pallas kernel here")


# speedup vs baseline: 1.1058x; 1.1058x over previous
import jax
import jax.numpy as jnp
from jax.experimental import pallas as pl
from jax.experimental.pallas import tpu as pltpu

_N = 50000
_E = 800000
_C = 121
_TEMP = 0.2
_BLK = 2000


def _loss_body(pred_ref, ty_ref, mf_ref, acc_ref):
    i = pl.program_id(0)

    @pl.when(i == 0)
    def _():
        acc_ref[...] = jnp.zeros_like(acc_ref)

    z = pred_ref[...]
    ty = ty_ref[...]
    mf = mf_ref[...]
    bce = jnp.maximum(z, 0.0) - z * ty + jnp.log1p(jnp.exp(-jnp.abs(z)))
    s = jnp.stack([(bce * mf).sum(), mf.sum()]).reshape(1, 2)
    acc_ref[...] += s


def _masked_bce_sums(pred, true_y, maskf):
    grid = _N // _BLK
    return pl.pallas_call(
        _loss_body,
        grid=(grid,),
        in_specs=[
            pl.BlockSpec((_BLK, _C), lambda i: (i, 0)),
            pl.BlockSpec((_BLK, _C), lambda i: (i, 0)),
            pl.BlockSpec((_BLK, 1), lambda i: (i, 0)),
        ],
        out_specs=pl.BlockSpec((1, 2), lambda i: (0, 0)),
        out_shape=jax.ShapeDtypeStruct((1, 2), jnp.float32),
    )(pred, true_y, maskf)


def kernel(x, edge_index, true_y, mask,
           W_proj_gate, b_proj_gate, a_src_gate, a_trg_gate,
           W_skip, b_skip, W_cheb0, W_cheb1, b_cheb,
           W_gat, a_src_gat, a_trg_gat, b_gat, gumbel_noise):
    src = edge_index[0]
    trg = edge_index[1]

    H = a_src_gate.shape[1]
    F = a_src_gate.shape[2]
    proj = (x @ W_proj_gate + b_proj_gate).reshape(-1, H, F)
    s_src = (proj * a_src_gate).sum(-1)
    s_trg = (proj * a_trg_gate).sum(-1)
    e_gate = (s_src[src] + s_trg[trg]).mean(-1)
    p = jax.nn.sigmoid(e_gate)
    z0 = (p + gumbel_noise[:, 0]) / _TEMP
    z1 = ((1.0 - p) + gumbel_noise[:, 1]) / _TEMP
    m = jnp.maximum(z0, z1)
    y0 = jnp.exp(z0 - m)
    y1 = jnp.exp(z1 - m)
    s = y0 + y1
    gate = (y0 / s >= y1 / s).astype(jnp.float32)

    x_skip = jax.nn.relu(x @ W_skip + b_skip)

    w = gate
    deg = jax.ops.segment_sum(w, src, num_segments=_N)
    dis = jnp.where(deg > 0, 1.0 / jnp.sqrt(jnp.maximum(deg, 1e-12)), 0.0)
    norm = -dis[src] * w * dis[trg]
    Tx1 = jax.ops.segment_sum(norm[:, None] * x[src], trg, num_segments=_N)
    x_global = jax.nn.relu(x @ W_cheb0 + Tx1 @ W_cheb1 + b_cheb)

    h = x_global + x_skip
    hp = h @ W_gat
    es = hp @ a_src_gat
    et = hp @ a_trg_gat
    e_att = jax.nn.leaky_relu(es[src] + et[trg], negative_slope=0.2)
    ex = jnp.exp(e_att)
    denom = jax.ops.segment_sum(ex, trg, num_segments=_N)
    att = ex / (denom[trg] + 1e-16)
    att = att * gate
    pred = jax.ops.segment_sum(att[:, None] * hp[src], trg, num_segments=_N) + b_gat

    sums = _masked_bce_sums(pred, true_y, mask.astype(jnp.float32)[:, None])
    pred_loss = sums[0, 0] / (sums[0, 1] * _C) + 2.0 * gate.sum() / _E
    return (pred_loss, pred)


# speedup vs baseline: 8.4939x; 7.6809x over previous
import jax
import jax.numpy as jnp
from jax import lax
from jax.experimental import pallas as pl
from jax.experimental.pallas import tpu as pltpu
from jax.experimental.pallas import tpu_sc as plsc

_N = 50000
_E = 800000
_C = 121
_TEMP = 0.2

_NP = 50176
_EP = 819200
_EW = _EP // 32
_CH = 1600
_NCH = _EW // _CH
_NT = _NP // 16

_BLK = 2000

_f32 = jnp.float32
_i32 = jnp.int32

def _mk_mesh():
    return plsc.VectorSubcoreMesh(core_axis_name="c", subcore_axis_name="s",
                                  num_cores=2, num_subcores=16)


def _gate_body(src_h, trg_h, g0_h, g1_h, ub_h, vb_h, zeros_h,
               gate_h, degp_h,
               ub_v, vb_v, srcb, trgb, g0b, g1b, gateb, nb, deg_sh):
    c = lax.axis_index("c")
    s = lax.axis_index("s")
    w = c * 16 + s
    base = w * _EW
    pltpu.sync_copy(ub_h, ub_v)
    pltpu.sync_copy(vb_h, vb_v)
    pltpu.sync_copy(zeros_h, nb)
    pltpu.sync_copy(nb, deg_sh.at[pl.ds(s * _NT, _NT)])
    plsc.subcore_barrier()

    def chunk(ci, carry):
        off = pl.multiple_of(base + ci * _CH, _CH)
        pltpu.sync_copy(src_h.at[pl.ds(off, _CH)], srcb)
        pltpu.sync_copy(trg_h.at[pl.ds(off, _CH)], trgb)
        pltpu.sync_copy(g0_h.at[pl.ds(off, _CH)], g0b)
        pltpu.sync_copy(g1_h.at[pl.ds(off, _CH)], g1b)

        def grp(j, inner):
            jo = pl.multiple_of(j * 16, 16)
            sidx = srcb[pl.ds(jo, 16)]
            tidx = trgb[pl.ds(jo, 16)]
            a = plsc.load_gather(ub_v, [sidx])
            b = plsc.load_gather(vb_v, [tidx])
            e = (a + b) * 0.5
            p = 1.0 / (1.0 + jnp.exp(-e))
            z0 = (p + g0b[pl.ds(jo, 16)]) / _TEMP
            z1 = ((1.0 - p) + g1b[pl.ds(jo, 16)]) / _TEMP
            m = jnp.maximum(z0, z1)
            y0 = jnp.exp(z0 - m)
            y1 = jnp.exp(z1 - m)
            ss = y0 + y1
            g = jnp.where(y0 / ss >= y1 / ss, 1.0, 0.0).astype(_f32)
            gid = off + jo + lax.iota(_i32, 16)
            g = jnp.where(gid < _E, g, 0.0)
            gateb[pl.ds(jo, 16)] = g
            return inner

        lax.fori_loop(0, _CH // 16, grp, 0)
        pltpu.sync_copy(gateb, gate_h.at[pl.ds(off, _CH)])
        pltpu.sync_copy(gateb, deg_sh.at[srcb], add=True)
        return carry

    lax.fori_loop(0, _NCH, chunk, 0)
    plsc.subcore_barrier()
    pltpu.sync_copy(deg_sh.at[pl.ds(s * _NT, _NT)], nb)
    pltpu.sync_copy(nb, degp_h.at[pl.ds(c * _NP + s * _NT, _NT)])


def _sc_gate(src_p, trg_p, g0_p, g1_p, ub, vb, zeros1):
    return pl.kernel(
        _gate_body,
        out_type=(jax.ShapeDtypeStruct((_EP,), _f32),
                  jax.ShapeDtypeStruct((2 * _NP,), _f32)),
        mesh=_mk_mesh(),
        compiler_params=pltpu.CompilerParams(needs_layout_passes=False, use_tc_tiling_on_sc=False),
        scratch_types=[
            pltpu.VMEM((_NP,), _f32),
            pltpu.VMEM((_NP,), _f32),
            pltpu.VMEM((_CH,), _i32),
            pltpu.VMEM((_CH,), _i32),
            pltpu.VMEM((_CH,), _f32),
            pltpu.VMEM((_CH,), _f32),
            pltpu.VMEM((_CH,), _f32),
            pltpu.VMEM((_NT,), _f32),
            pltpu.VMEM_SHARED((_NP,), _f32),
        ],
    )(src_p, trg_p, g0_p, g1_p, ub, vb, zeros1)


_CH2 = 256
_ET = _EP // 16
_NC2 = _ET // _CH2
_PC = 8
_NTP = _NT // _PC

def _cheb_body(src_h, trg_h, gate_h, xt2_h, zeros_h,
               tx0_h, tx1_h,
               srcb, trgb, gateb, binb, gidxb, rv, nb, acc_sh, sem):
    c = lax.axis_index("c")
    s = lax.axis_index("s")

    pltpu.sync_copy(zeros_h, nb)
    for p in range(_PC):
        pltpu.sync_copy(nb, acc_sh.at[pl.ds(s * _NT + p * _NTP, _NTP), :])
    plsc.subcore_barrier()

    def chunk(ci, carry):
        off = pl.multiple_of(s * _ET + ci * _CH2, _CH2)
        pltpu.sync_copy(src_h.at[pl.ds(off, _CH2)], srcb)
        pltpu.sync_copy(trg_h.at[pl.ds(off, _CH2)], trgb)
        pltpu.sync_copy(gate_h.at[pl.ds(off, _CH2)], gateb)

        def grp(j, inner):
            jo = pl.multiple_of(j * 16, 16)
            sv = srcb[pl.ds(jo, 16)]
            tv = trgb[pl.ds(jo, 16)]
            gv = gateb[pl.ds(jo, 16)]
            binb[pl.ds(jo, 16)] = jnp.where(gv > 0.0, tv, _N)
            gidxb[pl.ds(jo, 16)] = sv + c * _NP
            return inner

        lax.fori_loop(0, _CH2 // 16, grp, 0)
        pltpu.async_copy(xt2_h.at[gidxb], rv, sem).wait()
        pltpu.sync_copy(rv, acc_sh.at[binb], add=True)
        return carry

    lax.fori_loop(0, _NC2, chunk, 0)
    plsc.subcore_barrier()
    for p in range(_PC):
        r0 = s * _NT + p * _NTP
        pltpu.sync_copy(acc_sh.at[pl.ds(r0, _NTP), :], nb)

        @pl.when(c == 0)
        def _():
            pltpu.sync_copy(nb, tx0_h.at[pl.ds(r0, _NTP), :])

        @pl.when(c == 1)
        def _():
            pltpu.sync_copy(nb, tx1_h.at[pl.ds(r0, _NTP), :])


def _sc_cheb(src_p, trg_p, gate_p, xt2, zeros2):
    return pl.kernel(
        _cheb_body,
        out_type=(jax.ShapeDtypeStruct((_NP, 32), _f32),
                  jax.ShapeDtypeStruct((_NP, 32), _f32)),
        mesh=_mk_mesh(),
        compiler_params=pltpu.CompilerParams(needs_layout_passes=False, use_tc_tiling_on_sc=False),
        scratch_types=[
            pltpu.VMEM((_CH2,), _i32),
            pltpu.VMEM((_CH2,), _i32),
            pltpu.VMEM((_CH2,), _f32),
            pltpu.VMEM((_CH2,), _i32),
            pltpu.VMEM((_CH2,), _i32),
            pltpu.VMEM((_CH2, 32), _f32),
            pltpu.VMEM((_NTP, 32), _f32),
            pltpu.VMEM_SHARED((_NP, 32), _f32),
            pltpu.SemaphoreType.DMA,
        ],
    )(src_p, trg_p, gate_p, xt2, zeros2)


_N2 = 2 * _NP
_AR = 102400
_ART = _AR // 16
_ARTP = _ART // _PC


def _att_body(src_h, trg_h, gate_h, es_h, et_h, zeros_h,
              daccp_h, bin4_h, gidx4_h,
              es_v, et_v, srcb, trgb, gateb, valb, binb, gidxb, nb, dacc_sh):
    c = lax.axis_index("c")
    s = lax.axis_index("s")
    w = c * 16 + s
    base = w * _EW
    pltpu.sync_copy(es_h, es_v)
    pltpu.sync_copy(et_h, et_v)
    pltpu.sync_copy(zeros_h, nb)
    pltpu.sync_copy(nb, dacc_sh.at[pl.ds(s * (_N2 // 16), _N2 // 16)])
    plsc.subcore_barrier()

    def chunk(ci, carry):
        off = pl.multiple_of(base + ci * _CH, _CH)
        pltpu.sync_copy(src_h.at[pl.ds(off, _CH)], srcb)
        pltpu.sync_copy(trg_h.at[pl.ds(off, _CH)], trgb)
        pltpu.sync_copy(gate_h.at[pl.ds(off, _CH)], gateb)

        def grp(j, inner):
            jo = pl.multiple_of(j * 16, 16)
            sv = srcb[pl.ds(jo, 16)]
            tv = trgb[pl.ds(jo, 16)]
            gv = gateb[pl.ds(jo, 16)]
            a = plsc.load_gather(es_v, [sv])
            b = plsc.load_gather(et_v, [tv])
            pos = (a + b) >= 0.0
            val = jnp.exp(jnp.where(pos, a, 0.2 * a))
            gid = off + jo + lax.iota(_i32, 16)
            val = jnp.where(gid < _E, val, 0.0)
            bofs = jnp.where(pos, 0, _NP)
            valb[pl.ds(jo, 16)] = val
            binb[pl.ds(jo, 16)] = tv + bofs
            gidxb[pl.ds(jo, 16)] = sv + bofs
            return inner

        lax.fori_loop(0, _CH // 16, grp, 0)
        pltpu.sync_copy(valb, dacc_sh.at[binb], add=True)
        pltpu.sync_copy(gidxb, gidx4_h.at[pl.ds(off, _CH)])

        def grp2(j, inner):
            jo = pl.multiple_of(j * 16, 16)
            gv = gateb[pl.ds(jo, 16)]
            bv = binb[pl.ds(jo, 16)]
            binb[pl.ds(jo, 16)] = jnp.where(gv > 0.0, bv, _N2)
            return inner

        lax.fori_loop(0, _CH // 16, grp2, 0)
        pltpu.sync_copy(binb, bin4_h.at[pl.ds(off, _CH)])
        return carry

    lax.fori_loop(0, _NCH, chunk, 0)
    plsc.subcore_barrier()
    pltpu.sync_copy(dacc_sh.at[pl.ds(s * (_N2 // 16), _N2 // 16)], nb)
    pltpu.sync_copy(nb, daccp_h.at[pl.ds(c * _N2 + s * (_N2 // 16), _N2 // 16)])


def _sc_att(src_p, trg_p, gate_p, es, et, zeros3):
    return pl.kernel(
        _att_body,
        out_type=(jax.ShapeDtypeStruct((2 * _N2,), _f32),
                  jax.ShapeDtypeStruct((_EP,), _i32),
                  jax.ShapeDtypeStruct((_EP,), _i32)),
        mesh=_mk_mesh(),
        compiler_params=pltpu.CompilerParams(needs_layout_passes=False, use_tc_tiling_on_sc=False),
        scratch_types=[
            pltpu.VMEM((_NP,), _f32),
            pltpu.VMEM((_NP,), _f32),
            pltpu.VMEM((_CH,), _i32),
            pltpu.VMEM((_CH,), _i32),
            pltpu.VMEM((_CH,), _f32),
            pltpu.VMEM((_CH,), _f32),
            pltpu.VMEM((_CH,), _i32),
            pltpu.VMEM((_CH,), _i32),
            pltpu.VMEM((_N2 // 16,), _f32),
            pltpu.VMEM_SHARED((_N2,), _f32),
        ],
    )(src_p, trg_p, gate_p, es, et, zeros3)


def _pred_body(gidx4_h, bin4_h, hpt_h, zeros_h,
               pacc_h,
               gidxb, binb, rv, nb, acc_sh, sem):
    c = lax.axis_index("c")
    s = lax.axis_index("s")

    for r in range(4):
        b = r * 2 + c
        pltpu.sync_copy(zeros_h, nb)
        for p in range(_PC):
            pltpu.sync_copy(
                nb, acc_sh.at[pl.ds(s * _ART + p * _ARTP, _ARTP), :])
        plsc.subcore_barrier()

        def chunk(ci, carry, b=b):
            off = pl.multiple_of(s * _ET + ci * _CH2, _CH2)
            pltpu.sync_copy(gidx4_h.at[pl.ds(off, _CH2)], gidxb)
            pltpu.sync_copy(bin4_h.at[pl.ds(off, _CH2)], binb)

            def grp(j, inner):
                jo = pl.multiple_of(j * 16, 16)
                gidxb[pl.ds(jo, 16)] = gidxb[pl.ds(jo, 16)] + b * _N2
                return inner

            lax.fori_loop(0, _CH2 // 16, grp, 0)
            pltpu.async_copy(hpt_h.at[gidxb], rv, sem).wait()
            pltpu.sync_copy(rv, acc_sh.at[binb], add=True)
            return carry

        lax.fori_loop(0, _NC2, chunk, 0)
        plsc.subcore_barrier()
        for p in range(_PC):
            r0 = s * _ART + p * _ARTP
            pltpu.sync_copy(acc_sh.at[pl.ds(r0, _ARTP), :], nb)
            pltpu.sync_copy(nb, pacc_h.at[pl.ds(b * _AR + r0, _ARTP), :])


def _sc_pred(gidx4_p, bin4_p, hpt, zeros4):
    return pl.kernel(
        _pred_body,
        out_type=jax.ShapeDtypeStruct((8 * _AR, 16), _f32),
        mesh=_mk_mesh(),
        compiler_params=pltpu.CompilerParams(needs_layout_passes=False, use_tc_tiling_on_sc=False),
        scratch_types=[
            pltpu.VMEM((_CH2,), _i32),
            pltpu.VMEM((_CH2,), _i32),
            pltpu.VMEM((_CH2, 16), _f32),
            pltpu.VMEM((_ARTP, 16), _f32),
            pltpu.VMEM_SHARED((_AR, 16), _f32),
            pltpu.SemaphoreType.DMA,
        ],
    )(gidx4_p, bin4_p, hpt, zeros4)


def _loss_body(pred_ref, ty_ref, mf_ref, acc_ref):
    i = pl.program_id(0)

    @pl.when(i == 0)
    def _():
        acc_ref[...] = jnp.zeros_like(acc_ref)

    z = pred_ref[...]
    ty = ty_ref[...]
    mf = mf_ref[...]
    bce = jnp.maximum(z, 0.0) - z * ty + jnp.log1p(jnp.exp(-jnp.abs(z)))
    acc_ref[...] += jnp.stack([(bce * mf).sum(), mf.sum()]).reshape(1, 2)


def _masked_bce_sums(pred, true_y, maskf):
    return pl.pallas_call(
        _loss_body,
        grid=(_N // _BLK,),
        in_specs=[
            pl.BlockSpec((_BLK, _C), lambda i: (i, 0)),
            pl.BlockSpec((_BLK, _C), lambda i: (i, 0)),
            pl.BlockSpec((_BLK, 1), lambda i: (i, 0)),
        ],
        out_specs=pl.BlockSpec((1, 2), lambda i: (0, 0)),
        out_shape=jax.ShapeDtypeStruct((1, 2), _f32),
    )(pred, true_y, maskf)


def kernel(x, edge_index, true_y, mask,
           W_proj_gate, b_proj_gate, a_src_gate, a_trg_gate,
           W_skip, b_skip, W_cheb0, W_cheb1, b_cheb,
           W_gat, a_src_gat, a_trg_gat, b_gat, gumbel_noise):
    src = edge_index[0]
    trg = edge_index[1]
    src_p = jnp.pad(src, (0, _EP - _E))
    trg_p = jnp.pad(trg, (0, _EP - _E))
    g0_p = jnp.pad(gumbel_noise[:, 0], (0, _EP - _E))
    g1_p = jnp.pad(gumbel_noise[:, 1], (0, _EP - _E))

    H = a_src_gate.shape[1]
    F = a_src_gate.shape[2]
    proj = (x @ W_proj_gate + b_proj_gate).reshape(-1, H, F)
    s_src = (proj * a_src_gate).sum(-1)
    s_trg = (proj * a_trg_gate).sum(-1)
    ub = jnp.pad(s_src[:, 0] + s_src[:, 1], (0, _NP - _N))
    vb = jnp.pad(s_trg[:, 0] + s_trg[:, 1], (0, _NP - _N))

    zeros1 = jnp.zeros((_NT,), _f32)
    gate_p, degp = _sc_gate(src_p, trg_p, g0_p, g1_p, ub, vb, zeros1)
    deg = degp[:_N] + degp[_NP:_NP + _N]
    gate_sum = jnp.sum(deg)

    x_skip = jax.nn.relu(x @ W_skip + b_skip)

    dis = jnp.where(deg > 0, 1.0 / jnp.sqrt(jnp.maximum(deg, 1e-12)), 0.0)

    xt = x * dis[:, None]
    xt_pad = jnp.pad(xt, ((0, _NP - _N), (0, 0)))
    xt2 = jnp.concatenate([xt_pad[:, :32], xt_pad[:, 32:]], axis=0)
    zeros2 = jnp.zeros((_NTP, 32), _f32)
    tx0, tx1 = _sc_cheb(src_p, trg_p, gate_p, xt2, zeros2)
    Tx1 = -dis[:, None] * jnp.concatenate([tx0[:_N], tx1[:_N]], axis=1)

    x_global = jax.nn.relu(x @ W_cheb0 + Tx1 @ W_cheb1 + b_cheb)

    h = x_global + x_skip
    hp = h @ W_gat
    es = (hp * a_src_gat).sum(-1)
    et = (hp * a_trg_gat).sum(-1)

    es_pad = jnp.pad(es, (0, _NP - _N))
    et_pad = jnp.pad(et, (0, _NP - _N))
    zeros3 = jnp.zeros((_N2 // 16,), _f32)
    daccp, bin4_p, gidx4_p = _sc_att(src_p, trg_p, gate_p, es_pad, et_pad,
                                     zeros3)
    dacc = daccp[:_N2] + daccp[_N2:]
    B = jnp.exp(et)
    D = jnp.exp(0.2 * et)
    denom = B * dacc[:_N] + D * dacc[_NP:_NP + _N]

    hp_pad = jnp.pad(hp, ((0, _NP - _N), (0, 128 - _C)))
    A = jnp.exp(jnp.pad(es, (0, _NP - _N)))
    Cs = jnp.exp(0.2 * jnp.pad(es, (0, _NP - _N)))
    hpA = hp_pad * A[:, None]
    hpC = hp_pad * Cs[:, None]
    hpt = jnp.stack(
        [jnp.stack([hpA[:, 16 * b:16 * b + 16], hpC[:, 16 * b:16 * b + 16]])
         for b in range(8)]).reshape(8 * _N2, 16)
    zeros4 = jnp.zeros((_ARTP, 16), _f32)
    pacc = _sc_pred(gidx4_p, bin4_p, hpt, zeros4).reshape(8, _AR, 16)

    P = jnp.concatenate([pacc[b, :_N, :] for b in range(8)], axis=1)
    Q = jnp.concatenate([pacc[b, _NP:_NP + _N, :] for b in range(8)], axis=1)
    pred_pad = (B[:, None] * P + D[:, None] * Q) / (denom[:, None] + 1e-16)
    pred = pred_pad[:, :_C] + b_gat

    sums = _masked_bce_sums(pred, true_y, mask.astype(_f32)[:, None])
    pred_loss = sums[0, 0] / (sums[0, 1] * _C) + 2.0 * gate_sum / _E
    return (pred_loss, pred)
